# baseline (speedup 1.0000x reference)
import functools

import jax
import jax.numpy as jnp
from jax import lax
from jax.experimental import pallas as pl
from jax.experimental.pallas import tpu as pltpu
from jax.experimental.pallas import tpu_sc as plsc

N_TOK = 16384
N_EXP = 64
K = 8
NC = 2
NS = 16
NW = NC * NS
TPW = N_TOK // NW
SPW = TPW * K
GRID = 8
BT = N_TOK // GRID
CPB = BT // TPW


def _tc_body(logits_ref, probs_ref, scores_ref, assign_ref, assign_t_ref,
             bhist_ref):
    x = logits_ref[:]
    m = jnp.max(x, axis=1, keepdims=True)
    e = jnp.exp(x - m)
    p = e / jnp.sum(e, axis=1, keepdims=True)
    probs_ref[:] = p
    lane_rev = (
        (N_EXP - 1) - lax.broadcasted_iota(jnp.int32, (BT, N_EXP), 1)
    ).astype(jnp.float32)
    work = p
    cols = []
    for k in range(K):
        mk = jnp.max(work, axis=1, keepdims=True)
        cand = jnp.where(work == mk, lane_rev, -1.0)
        mrev = jnp.max(cand, axis=1, keepdims=True)
        work = jnp.where(cand == mrev, -1.0, work)
        idx = ((N_EXP - 1.0) - mrev).astype(jnp.int32)
        cols.append(idx)
        scores_ref[:, k:k + 1] = mk
        assign_ref[:, k:k + 1] = idx
    assign_t_ref[:] = jnp.transpose(jnp.concatenate(cols, axis=1))
    sel_all = (work < 0.0).astype(jnp.int32)
    for g in range(CPB):
        bhist_ref[g, 0, :] = jnp.sum(
            sel_all[g * TPW:(g + 1) * TPW], axis=0
        )


_tc_call = pl.pallas_call(
    _tc_body,
    grid=(GRID,),
    in_specs=[pl.BlockSpec((BT, N_EXP), lambda i: (i, 0))],
    out_specs=[
        pl.BlockSpec((BT, N_EXP), lambda i: (i, 0)),
        pl.BlockSpec((BT, K), lambda i: (i, 0)),
        pl.BlockSpec((BT, K), lambda i: (i, 0)),
        pl.BlockSpec((K, BT), lambda i: (0, i)),
        pl.BlockSpec((CPB, 1, N_EXP), lambda i: (i, 0, 0)),
    ],
    out_shape=[
        jax.ShapeDtypeStruct((N_TOK, N_EXP), jnp.float32),
        jax.ShapeDtypeStruct((N_TOK, K), jnp.float32),
        jax.ShapeDtypeStruct((N_TOK, K), jnp.int32),
        jax.ShapeDtypeStruct((K, N_TOK), jnp.int32),
        jax.ShapeDtypeStruct((NW, 1, N_EXP), jnp.int32),
    ],
)


def _tc2_body(offs_t_ref, offs_ref):
    offs_ref[:] = jnp.transpose(offs_t_ref[:])


_tc2_call = pl.pallas_call(
    _tc2_body,
    grid=(GRID,),
    in_specs=[pl.BlockSpec((K, BT), lambda i: (0, i))],
    out_specs=[pl.BlockSpec((BT, K), lambda i: (i, 0))],
    out_shape=[jax.ShapeDtypeStruct((N_TOK, K), jnp.int32)],
)


def _sc_body(assign_t_hbm, bhist_hbm, counts_out, offs_t_out,
             bh_v, a_v, o_v, hist_v, tot_v):
    c = lax.axis_index("c")
    s = lax.axis_index("s")
    wid = s * NC + c
    pltpu.sync_copy(bhist_hbm, bh_v)
    for k in range(K):
        pltpu.sync_copy(
            assign_t_hbm.at[pl.ds(k * N_TOK + wid * TPW, TPW)],
            a_v.at[pl.ds(k * TPW, TPW)],
        )
    for j in range(N_EXP // 16):
        acc = jnp.zeros((16,), jnp.int32)
        tot = jnp.zeros((16,), jnp.int32)
        for u in range(NW):
            v = bh_v[pl.ds(u * N_EXP + j * 16, 16)]
            pre = (jnp.int32(u) < wid).astype(jnp.int32)
            acc = acc + v * pre
            tot = tot + v
        hist_v[pl.ds(j * 16, 16)] = acc
        tot_v[pl.ds(j * 16, 16)] = tot

    lane = lax.broadcasted_iota(jnp.int32, (16,), 0)
    mask8 = lane < K
    ones = jnp.ones((16,), jnp.int32)
    lane_tpw = lane * TPW

    def tok_body(t, carry):
        slot = lane_tpw + t
        idx = plsc.load_gather(a_v, [slot], mask=mask8)
        g = plsc.load_gather(hist_v, [idx], mask=mask8)
        plsc.addupdate_scatter(hist_v, [idx], ones, mask=mask8)
        plsc.store_scatter(o_v, [slot], g, mask=mask8)
        return carry

    lax.fori_loop(0, TPW, tok_body, 0)
    for k in range(K):
        pltpu.sync_copy(
            o_v.at[pl.ds(k * TPW, TPW)],
            offs_t_out.at[pl.ds(k * N_TOK + wid * TPW, TPW)],
        )

    @pl.when(wid == 0)
    def _():
        pltpu.sync_copy(tot_v, counts_out)


@functools.cache
def _sc_call():
    mesh = plsc.VectorSubcoreMesh(
        core_axis_name="c", subcore_axis_name="s", num_cores=NC, num_subcores=NS
    )
    return functools.partial(
        pl.kernel,
        mesh=mesh,
        compiler_params=pltpu.CompilerParams(needs_layout_passes=False),
        out_type=[
            jax.ShapeDtypeStruct((N_EXP,), jnp.int32),
            jax.ShapeDtypeStruct((K * N_TOK,), jnp.int32),
        ],
        scratch_types=[
            pltpu.VMEM((NW * N_EXP,), jnp.int32),
            pltpu.VMEM((SPW,), jnp.int32),
            pltpu.VMEM((SPW,), jnp.int32),
            pltpu.VMEM((N_EXP,), jnp.int32),
            pltpu.VMEM((N_EXP,), jnp.int32),
        ],
    )(_sc_body)


def kernel(expert_counts, assignments, offsets, logits):
    probs, scores, assign, assign_t, bhist = _tc_call(logits)
    counts, offs_t = _sc_call()(assign_t.reshape(-1), bhist.reshape(-1))
    offs = _tc2_call(offs_t.reshape(K, N_TOK))[0]
    return counts, scores, assign, offs, probs


# speedup vs baseline: 1.0347x; 1.0347x over previous
import functools

import jax
import jax.numpy as jnp
from jax import lax
from jax.experimental import pallas as pl
from jax.experimental.pallas import tpu as pltpu
from jax.experimental.pallas import tpu_sc as plsc

N_TOK = 16384
N_EXP = 64
K = 8
NC = 2
NS = 16
NW = NC * NS
TPW = N_TOK // NW
SPW = TPW * K
GRID = 8
BT = N_TOK // GRID
CPB = BT // TPW


def _tc_body(logits_ref, probs_ref, scores_ref, assign_ref, assign_t_ref,
             bhist_ref):
    x = logits_ref[:]
    m = jnp.max(x, axis=1, keepdims=True)
    e = jnp.exp(x - m)
    p = e / jnp.sum(e, axis=1, keepdims=True)
    probs_ref[:] = p
    lane_rev = (
        (N_EXP - 1) - lax.broadcasted_iota(jnp.int32, (BT, N_EXP), 1)
    ).astype(jnp.float32)
    work = p
    cols = []
    for k in range(K):
        mk = jnp.max(work, axis=1, keepdims=True)
        cand = jnp.where(work == mk, lane_rev, -1.0)
        mrev = jnp.max(cand, axis=1, keepdims=True)
        work = jnp.where(cand == mrev, -1.0, work)
        idx = ((N_EXP - 1.0) - mrev).astype(jnp.int32)
        cols.append(idx)
        scores_ref[:, k:k + 1] = mk
        assign_ref[:, k:k + 1] = idx
    assign_t_ref[:] = jnp.transpose(jnp.concatenate(cols, axis=1))
    sel_all = (work < 0.0).astype(jnp.int32)
    for g in range(CPB):
        bhist_ref[g, 0, :] = jnp.sum(
            sel_all[g * TPW:(g + 1) * TPW], axis=0
        )


_tc_call = pl.pallas_call(
    _tc_body,
    grid=(GRID,),
    in_specs=[pl.BlockSpec((BT, N_EXP), lambda i: (i, 0))],
    out_specs=[
        pl.BlockSpec((BT, N_EXP), lambda i: (i, 0)),
        pl.BlockSpec((BT, K), lambda i: (i, 0)),
        pl.BlockSpec((BT, K), lambda i: (i, 0)),
        pl.BlockSpec((K, BT), lambda i: (0, i)),
        pl.BlockSpec((CPB, 1, N_EXP), lambda i: (i, 0, 0)),
    ],
    out_shape=[
        jax.ShapeDtypeStruct((N_TOK, N_EXP), jnp.float32),
        jax.ShapeDtypeStruct((N_TOK, K), jnp.float32),
        jax.ShapeDtypeStruct((N_TOK, K), jnp.int32),
        jax.ShapeDtypeStruct((K, N_TOK), jnp.int32),
        jax.ShapeDtypeStruct((NW, 1, N_EXP), jnp.int32),
    ],
)


def _tc2_body(offs_t_ref, offs_ref):
    offs_ref[:] = jnp.transpose(offs_t_ref[:])


_tc2_call = pl.pallas_call(
    _tc2_body,
    grid=(GRID,),
    in_specs=[pl.BlockSpec((K, BT), lambda i: (0, i))],
    out_specs=[pl.BlockSpec((BT, K), lambda i: (i, 0))],
    out_shape=[jax.ShapeDtypeStruct((N_TOK, K), jnp.int32)],
)


def _sc_body(assign_t_hbm, bhist_hbm, counts_out, offs_t_out,
             bh_v, a_v, o_v, hist_v, tot_v):
    c = lax.axis_index("c")
    s = lax.axis_index("s")
    wid = s * NC + c
    pltpu.sync_copy(bhist_hbm, bh_v)
    for k in range(K):
        pltpu.sync_copy(
            assign_t_hbm.at[k, pl.ds(wid * TPW, TPW)],
            a_v.at[pl.ds(k * TPW, TPW)],
        )
    for j in range(N_EXP // 16):
        acc = jnp.zeros((16,), jnp.int32)
        tot = jnp.zeros((16,), jnp.int32)
        for u in range(NW):
            v = bh_v[pl.ds(u * N_EXP + j * 16, 16)]
            pre = (jnp.int32(u) < wid).astype(jnp.int32)
            acc = acc + v * pre
            tot = tot + v
        hist_v[pl.ds(j * 16, 16)] = acc
        tot_v[pl.ds(j * 16, 16)] = tot

    lane = lax.broadcasted_iota(jnp.int32, (16,), 0)
    mask8 = lane < K
    ones = jnp.ones((16,), jnp.int32)
    lane_tpw = lane * TPW

    def tok_body(t, carry):
        slot = lane_tpw + t
        idx = plsc.load_gather(a_v, [slot], mask=mask8)
        g = plsc.load_gather(hist_v, [idx], mask=mask8)
        plsc.addupdate_scatter(hist_v, [idx], ones, mask=mask8)
        plsc.store_scatter(o_v, [slot], g, mask=mask8)
        return carry

    lax.fori_loop(0, TPW, tok_body, 0)
    for k in range(K):
        pltpu.sync_copy(
            o_v.at[pl.ds(k * TPW, TPW)],
            offs_t_out.at[k, pl.ds(wid * TPW, TPW)],
        )

    @pl.when(wid == 0)
    def _():
        pltpu.sync_copy(tot_v, counts_out)


@functools.cache
def _sc_call():
    mesh = plsc.VectorSubcoreMesh(
        core_axis_name="c", subcore_axis_name="s", num_cores=NC, num_subcores=NS
    )
    return functools.partial(
        pl.kernel,
        mesh=mesh,
        compiler_params=pltpu.CompilerParams(needs_layout_passes=False),
        out_type=[
            jax.ShapeDtypeStruct((N_EXP,), jnp.int32),
            jax.ShapeDtypeStruct((K, N_TOK), jnp.int32),
        ],
        scratch_types=[
            pltpu.VMEM((NW * N_EXP,), jnp.int32),
            pltpu.VMEM((SPW,), jnp.int32),
            pltpu.VMEM((SPW,), jnp.int32),
            pltpu.VMEM((N_EXP,), jnp.int32),
            pltpu.VMEM((N_EXP,), jnp.int32),
        ],
    )(_sc_body)


def kernel(expert_counts, assignments, offsets, logits):
    probs, scores, assign, assign_t, bhist = _tc_call(logits)
    counts, offs_t = _sc_call()(assign_t, bhist.reshape(-1))
    offs = _tc2_call(offs_t)[0]
    return counts, scores, assign, offs, probs
